# Initial kernel scaffold; baseline (speedup 1.0000x reference)
import jax
import jax.numpy as jnp
from jax.experimental import pallas as pl


def kernel(x, embeddings):
    raise NotImplementedError("write your pallas kernel here")


# speedup vs baseline: 1.1096x; 1.1096x over previous
import functools

import jax
import jax.numpy as jnp
from jax import lax
from jax.experimental import pallas as pl
from jax.experimental.pallas import tpu as pltpu
from jax.experimental.pallas import tpu_sc as plsc

IDX_MINOR = 128


@functools.cache
def _make_sc_gather(B: int, V: int, D: int):
    info = plsc.get_sparse_core_info()
    num_workers = info.num_cores * info.num_subcores
    rows_per_w = B // num_workers
    idxrows_per_w = rows_per_w // IDX_MINOR
    gathers_per_chunk = 20
    chunk = gathers_per_chunk * IDX_MINOR
    nchunk = rows_per_w // chunk
    assert chunk * nchunk == rows_per_w

    mesh = plsc.VectorSubcoreMesh(core_axis_name="c", subcore_axis_name="s")

    @functools.partial(
        pl.kernel,
        out_type=jax.ShapeDtypeStruct((B, D), jnp.float32),
        mesh=mesh,
        scratch_types=[
            pltpu.VMEM((idxrows_per_w, IDX_MINOR), jnp.int32),
            pltpu.VMEM((chunk, D), jnp.float32),
            pltpu.SemaphoreType.DMA,
        ],
        compiler_params=pltpu.CompilerParams(use_tc_tiling_on_sc=False),
    )
    def gather_kernel(idx_hbm, table_hbm, out_hbm, idx_v, rows_v, sem):
        wid = lax.axis_index("s") * info.num_cores + lax.axis_index("c")
        pltpu.sync_copy(
            idx_hbm.at[pl.ds(wid * idxrows_per_w, idxrows_per_w)], idx_v
        )
        out_base = wid * rows_per_w

        @pl.loop(0, nchunk)
        def _chunk(c):
            descs = []
            for j in range(gathers_per_chunk):
                irow = c * gathers_per_chunk + j
                descs.append(
                    pltpu.async_copy(
                        table_hbm.at[idx_v.at[irow]],
                        rows_v.at[pl.ds(j * IDX_MINOR, IDX_MINOR)],
                        sem,
                    )
                )
            for d in descs:
                d.wait()
            pltpu.sync_copy(
                rows_v, out_hbm.at[pl.ds(out_base + c * chunk, chunk)]
            )

    return gather_kernel


@jax.jit
def kernel(x, embeddings):
    N_, L_ = x.shape
    V, D = embeddings.shape
    B = N_ * L_
    idx = x.reshape(B // IDX_MINOR, IDX_MINOR).astype(jnp.int32)
    out = _make_sc_gather(B, V, D)(idx, embeddings)
    return out.reshape(N_, L_, D)
